# baseline (speedup 1.0000x reference)
import functools

import jax
import jax.numpy as jnp
from jax import lax
from jax.experimental import pallas as pl
from jax.experimental.pallas import tpu as pltpu
from jax.experimental.pallas import tpu_sc as plsc

N = 10000
E = 320000
HID = 128
NC = 2
NS = 16
L = 16
CHUNK = 128
TR = 160
SLAB = 8
ROWS_PER_TILE = 632
ACC_ROWS = NS * ROWS_PER_TILE
TRASH = N
WTAB = N + 128
DW_LEN = 3 * N
RB = 400
NB = N // RB


@functools.cache
def _make_sc_pass_a():
    return functools.partial(
        pl.kernel,
        mesh=plsc.VectorSubcoreMesh(core_axis_name="c", subcore_axis_name="s"),
        compiler_params=pltpu.CompilerParams(needs_layout_passes=False),
        out_type=[
            jax.ShapeDtypeStruct((3, ACC_ROWS, HID), jnp.float32),
            jax.ShapeDtypeStruct((3 * ACC_ROWS,), jnp.float32),
            jax.ShapeDtypeStruct((DW_LEN,), jnp.float32),
        ],
        scratch_types=[
            pltpu.VMEM((SLAB, CHUNK), jnp.int32),
            pltpu.VMEM((SLAB, CHUNK), jnp.int32),
            pltpu.VMEM((2, CHUNK, HID), jnp.float32),
            pltpu.VMEM((CHUNK,), jnp.float32),
            pltpu.VMEM((CHUNK,), jnp.float32),
            pltpu.VMEM((WTAB,), jnp.float32),
            pltpu.VMEM_SHARED((ACC_ROWS, HID), jnp.float32),
            pltpu.VMEM_SHARED((ACC_ROWS,), jnp.float32),
            pltpu.VMEM_SHARED((DW_LEN,), jnp.float32),
            pltpu.SemaphoreType.DMA,
            pltpu.SemaphoreType.DMA,
        ],
    )(_sc_a_body)


def _zero_acc_slice(z128, zbuf, acc_f, base):
    pltpu.sync_copy(z128, zbuf)
    nfull = ROWS_PER_TILE // CHUNK
    rem = ROWS_PER_TILE - nfull * CHUNK
    for k in range(nfull):
        pltpu.sync_copy(zbuf, acc_f.at[pl.ds(base + k * CHUNK, CHUNK)])
    if rem:
        pltpu.sync_copy(zbuf.at[pl.ds(0, rem)],
                        acc_f.at[pl.ds(base + nfull * CHUNK, rem)])


def _row_loop(table, gidx, sidx, gidx_v, sidx_v, bufs, acc_f, sems, v, s,
              nslab):
    def body(t, carry):
        pltpu.sync_copy(gidx.at[v, s, pl.ds(t * SLAB, SLAB)], gidx_v)
        pltpu.sync_copy(sidx.at[v, s, pl.ds(t * SLAB, SLAB)], sidx_v)
        cp = pltpu.async_copy(table.at[gidx_v.at[0]], bufs.at[0], sems[0])
        for j in range(SLAB):
            p = j % 2
            if j + 1 < SLAB:
                cp_next = pltpu.async_copy(table.at[gidx_v.at[j + 1]],
                                           bufs.at[1 - p], sems[1 - p])
            cp.wait()
            pltpu.sync_copy(bufs.at[p], acc_f.at[sidx_v.at[j]], add=True)
            if j + 1 < SLAB:
                cp = cp_next
        return carry

    lax.fori_loop(0, nslab, body, 0)


def _sc_a_body(table, w_tab, gidx, sidx, onesc, z128, zsc, zdw,
               out_f, out_bc, out_dw,
               gidx_v, sidx_v, bufs, vones, vw, w_v,
               acc_f, acc_bc, acc_dw, sem0, sem1):
    c = lax.axis_index("c")
    s = lax.axis_index("s")
    base = s * ROWS_PER_TILE
    sems = (sem0, sem1)
    nrow = jnp.where(c == 0, TR // SLAB, 0)
    nsc = jnp.where(c == 0, 0, TR // SLAB)
    pltpu.sync_copy(onesc, vones)

    @pl.when(jnp.logical_and(c == 1, s == 0))
    def _():
        pltpu.sync_copy(zdw, acc_dw)

    for v in range(3):
        pltpu.sync_copy(w_tab.at[pl.ds(v * WTAB, WTAB)], w_v)
        _zero_acc_slice(z128, bufs.at[0], acc_f, base)

        @pl.when(s == 0)
        def _():
            pltpu.sync_copy(zsc, acc_bc)

        plsc.subcore_barrier()
        _row_loop(table, gidx, sidx, gidx_v, sidx_v, bufs, acc_f,
                  sems, v, s, nrow)

        def body(t, carry):
            pltpu.sync_copy(gidx.at[v, s, pl.ds(t * SLAB, SLAB)], gidx_v)
            pltpu.sync_copy(sidx.at[v, s, pl.ds(t * SLAB, SLAB)], sidx_v)
            for j in range(SLAB):
                for k in range(CHUNK // L):
                    gi = sidx_v[j, pl.ds(k * L, L)]
                    vw[pl.ds(k * L, L)] = plsc.load_gather(w_v, [gi])
                pltpu.sync_copy(vones, acc_bc.at[sidx_v.at[j]], add=True)
                pltpu.sync_copy(vw, acc_dw.at[gidx_v.at[j]], add=True)
            return carry

        lax.fori_loop(0, nsc, body, 0)
        plsc.subcore_barrier()

        @pl.when(c == 0)
        def _():
            pltpu.sync_copy(acc_f.at[pl.ds(base, ROWS_PER_TILE)],
                            out_f.at[v, pl.ds(base, ROWS_PER_TILE)])

        @pl.when(jnp.logical_and(c == 1, s == 0))
        def _():
            pltpu.sync_copy(acc_bc,
                            out_bc.at[pl.ds(v * ACC_ROWS, ACC_ROWS)])

    @pl.when(jnp.logical_and(c == 1, s == 0))
    def _():
        pltpu.sync_copy(acc_dw, out_dw)


@functools.cache
def _make_sc_pass_b():
    return functools.partial(
        pl.kernel,
        mesh=plsc.VectorSubcoreMesh(core_axis_name="c", subcore_axis_name="s"),
        compiler_params=pltpu.CompilerParams(needs_layout_passes=False),
        out_type=[
            jax.ShapeDtypeStruct((3, ACC_ROWS, HID), jnp.float32),
        ],
        scratch_types=[
            pltpu.VMEM((SLAB, CHUNK), jnp.int32),
            pltpu.VMEM((SLAB, CHUNK), jnp.int32),
            pltpu.VMEM((2, CHUNK, HID), jnp.float32),
            pltpu.VMEM_SHARED((ACC_ROWS, HID), jnp.float32),
            pltpu.SemaphoreType.DMA,
            pltpu.SemaphoreType.DMA,
        ],
    )(_sc_b_body)


def _sc_b_body(table, gidx, sidx, z128, out_f,
               gidx_v, sidx_v, bufs, acc_f, sem0, sem1):
    c = lax.axis_index("c")
    s = lax.axis_index("s")
    base = s * ROWS_PER_TILE
    sems = (sem0, sem1)
    nrow = jnp.where(c == 0, TR // SLAB, 0)
    for v in range(3):
        _zero_acc_slice(z128, bufs.at[0], acc_f, base)
        plsc.subcore_barrier()
        _row_loop(table, gidx, sidx, gidx_v, sidx_v, bufs, acc_f,
                  sems, v, s, nrow)
        plsc.subcore_barrier()

        @pl.when(c == 0)
        def _():
            pltpu.sync_copy(acc_f.at[pl.ds(base, ROWS_PER_TILE)],
                            out_f.at[v, pl.ds(base, ROWS_PER_TILE)])


def _pre_body(x_ref, w_ref, out_ref):
    out_ref[...] = jnp.dot(x_ref[...], w_ref[0],
                           preferred_element_type=jnp.float32)


def _tc_pre(x, w_stacked):
    return pl.pallas_call(
        _pre_body,
        grid=(3, NB),
        in_specs=[
            pl.BlockSpec((RB, HID), lambda v, i: (i, 0)),
            pl.BlockSpec((1, HID, HID), lambda v, i: (v, 0, 0)),
        ],
        out_specs=pl.BlockSpec((RB, HID), lambda v, i: (v * NB + i, 0)),
        out_shape=jax.ShapeDtypeStruct((3 * N, HID), jnp.float32),
    )(x, w_stacked)


def _mid_body(accf_ref, accs_ref, w_ref, out_ref):
    a = accf_ref[0]
    bc = accs_ref[0]
    binv = jnp.where(bc > 0, 1.0 / bc, 0.0)
    out_ref[...] = (binv * w_ref[...]) * a


def _tc_mid(acc_f, acc_s, w_col):
    return pl.pallas_call(
        _mid_body,
        grid=(3, NB),
        in_specs=[
            pl.BlockSpec((1, RB, HID), lambda v, i: (v, i, 0)),
            pl.BlockSpec((1, RB, 1), lambda v, i: (v, i, 0)),
            pl.BlockSpec((RB, 1), lambda v, i: (v * NB + i, 0)),
        ],
        out_specs=pl.BlockSpec((RB, HID), lambda v, i: (v * NB + i, 0)),
        out_shape=jax.ShapeDtypeStruct((3 * N, HID), jnp.float32),
    )(acc_f, acc_s, w_col)


def _leaky(h):
    return jnp.where(h >= 0, h, 0.2 * h)


def _ln(h, g, b):
    mu = jnp.mean(h, axis=-1, keepdims=True)
    var = jnp.mean((h - mu) ** 2, axis=-1, keepdims=True)
    return (h - mu) * lax.rsqrt(var + 1e-5) * g + b


def _final_body(accf_ref, accs_ref, x_ref, bst_ref,
                wa1_ref, ba1_ref, wa2_ref, ba2_ref,
                wb_ref, bb_ref, gb_ref, beb_ref,
                wg1_ref, bg1_ref, wg2_ref, bg2_ref,
                wc1_ref, bc1_ref, gc_ref, bec_ref, wc2_ref, bc2_ref,
                out_ref):
    x = x_ref[...]
    f32 = jnp.float32
    zs = []
    for v in range(3):
        av = accf_ref[v]
        dw = accs_ref[v]
        dinv = jnp.where(dw > 0, 1.0 / dw, 0.0)
        zs.append(_leaky(dinv * av + bst_ref[v:v + 1]))
    scores = []
    for v in range(3):
        t = jnp.tanh(jnp.dot(zs[v], wa1_ref[...], preferred_element_type=f32)
                     + ba1_ref[...])
        scores.append(jnp.dot(t, wa2_ref[...], preferred_element_type=f32)
                      + ba2_ref[...])
    mx = jnp.maximum(scores[0], jnp.maximum(scores[1], scores[2]))
    es = [jnp.exp(sv - mx) for sv in scores]
    den = es[0] + es[1] + es[2]
    z_sp = (es[0] * zs[0] + es[1] * zs[1] + es[2] * zs[2]) / den
    zb = _leaky(_ln(jnp.dot(x, wb_ref[...], preferred_element_type=f32)
                    + bb_ref[...], gb_ref[...], beb_ref[...]))
    se = jnp.dot(z_sp, wg1_ref[0], preferred_element_type=f32) \
        + jnp.dot(zb, wg1_ref[1], preferred_element_type=f32) + bg1_ref[...]
    se = jnp.maximum(se, 0.0)
    gl = jnp.dot(se, wg2_ref[...], preferred_element_type=f32) + bg2_ref[...]
    g = 1.0 / (1.0 + jnp.exp(-gl))
    zf = g * z_sp + (1.0 - g) * zb
    h = jnp.dot(zf, wc1_ref[0], preferred_element_type=f32) \
        + jnp.dot(x, wc1_ref[1], preferred_element_type=f32) + bc1_ref[...]
    h = jnp.maximum(_ln(h, gc_ref[...], bec_ref[...]), 0.0)
    out_ref[...] = jnp.dot(h, wc2_ref[...], preferred_element_type=f32) \
        + bc2_ref[...]


def _tc_final(acc_f, acc_s, x, bst, wa1, ba1, wa2, ba2, wb, bb, gb, beb,
              wg1, bg1, wg2, bg2, wc1, bc1, gc, bec, wc2, bc2):
    def full(shape):
        return pl.BlockSpec(shape, lambda i: tuple(0 for _ in shape))
    return pl.pallas_call(
        _final_body,
        grid=(NB,),
        in_specs=[
            pl.BlockSpec((3, RB, HID), lambda i: (0, i, 0)),
            pl.BlockSpec((3, RB, 1), lambda i: (0, i, 0)),
            pl.BlockSpec((RB, HID), lambda i: (i, 0)),
            full(bst.shape), full(wa1.shape), full(ba1.shape),
            full(wa2.shape), full(ba2.shape),
            full(wb.shape), full(bb.shape), full(gb.shape), full(beb.shape),
            full(wg1.shape), full(bg1.shape), full(wg2.shape),
            full(bg2.shape),
            full(wc1.shape), full(bc1.shape), full(gc.shape), full(bec.shape),
            full(wc2.shape), full(bc2.shape),
        ],
        out_specs=pl.BlockSpec((RB, 1), lambda i: (i, 0)),
        out_shape=jax.ShapeDtypeStruct((N, 1), jnp.float32),
    )(acc_f, acc_s, x, bst, wa1, ba1, wa2, ba2, wb, bb, gb, beb,
      wg1, bg1, wg2, bg2, wc1, bc1, gc, bec, wc2, bc2)


def _pack_idx(rows, pad_value):
    cap = NS * TR * CHUNK
    p = jnp.full((3, cap - E), pad_value, dtype=jnp.int32)
    return jnp.concatenate([rows.astype(jnp.int32), p],
                           axis=1).reshape(3, NS, TR, CHUNK)


def kernel(x, e_auc, e_ip, e_dev, w_auc, w_ip, w_dev, W_auc, b_auc, W_ip,
           b_ip, W_dev, b_dev, Wa1, ba1, Wa2, ba2, Wb, bb, gb, beb, Wg1, bg1,
           Wg2, bg2, Wc1, bc1, gc, bec, Wc2, bc2):
    f32 = jnp.float32
    w_stacked = jnp.stack([W_auc, W_ip, W_dev])
    bst = jnp.stack([b_auc, b_ip, b_dev])
    ws = jnp.stack([w_auc, w_ip, w_dev])
    w_flat = ws.reshape(3 * N)
    w_tab = jnp.pad(ws, ((0, 0), (0, WTAB - N))).reshape(3 * WTAB)
    onesc = jnp.ones((CHUNK,), f32)
    z128 = jnp.zeros((CHUNK, HID), f32)
    zsc = jnp.zeros((ACC_ROWS,), f32)
    zdw = jnp.zeros((DW_LEN,), f32)

    ni = jnp.stack([e_auc[0], e_ip[0], e_dev[0]])
    ei = jnp.stack([e_auc[1], e_ip[1], e_dev[1]])
    voff = (jnp.arange(3, dtype=jnp.int32) * N)[:, None]
    g_a = _pack_idx(ni + voff, 0)
    s_a = _pack_idx(ei, TRASH)
    g_b = _pack_idx(ei + voff, 0)
    s_b = _pack_idx(ni, TRASH)

    xt = _tc_pre(x, w_stacked)
    accf_a, acc_bc, acc_dw = _make_sc_pass_a()(
        xt, w_tab, g_a, s_a, onesc, z128, zsc, zdw)
    m_t = _tc_mid(accf_a, acc_bc.reshape(3, ACC_ROWS, 1),
                  w_flat.reshape(3 * N, 1))
    accf_b, = _make_sc_pass_b()(m_t, g_b, s_b, z128)
    dw3 = acc_dw.reshape(3, N, 1)
    return _tc_final(
        accf_b, dw3, x, bst,
        Wa1, ba1.reshape(1, -1), Wa2, ba2.reshape(1, 1),
        Wb, bb.reshape(1, -1), gb.reshape(1, -1), beb.reshape(1, -1),
        Wg1.reshape(2, HID, HID // 2), bg1.reshape(1, -1), Wg2,
        bg2.reshape(1, -1),
        Wc1.reshape(2, HID, HID), bc1.reshape(1, -1), gc.reshape(1, -1),
        bec.reshape(1, -1), Wc2, bc2.reshape(1, 1))


# speedup vs baseline: 1.4556x; 1.4556x over previous
import functools

import jax
import jax.numpy as jnp
from jax import lax
from jax.experimental import pallas as pl
from jax.experimental.pallas import tpu as pltpu
from jax.experimental.pallas import tpu_sc as plsc

N = 10000
E = 320000
HID = 128
NC = 2
NS = 16
L = 16
CHUNK = 128
T0 = 136
T1 = 24
NCHUNK = max(T0, T1)
SLAB = 8
ROWS_PER_TILE = 632
ACC_ROWS = NS * ROWS_PER_TILE
TRASH = N
WTAB = N + 128
DW_LEN = 3 * N
RB = 400
NB = N // RB


@functools.cache
def _make_sc_pass_a():
    return functools.partial(
        pl.kernel,
        mesh=plsc.VectorSubcoreMesh(core_axis_name="c", subcore_axis_name="s"),
        compiler_params=pltpu.CompilerParams(needs_layout_passes=False),
        out_type=[
            jax.ShapeDtypeStruct((3, NC, ACC_ROWS, HID), jnp.float32),
            jax.ShapeDtypeStruct((3, NC, ACC_ROWS), jnp.float32),
            jax.ShapeDtypeStruct((NC, DW_LEN), jnp.float32),
        ],
        scratch_types=[
            pltpu.VMEM((SLAB, CHUNK), jnp.int32),
            pltpu.VMEM((SLAB, CHUNK), jnp.int32),
            pltpu.VMEM((2, CHUNK, HID), jnp.float32),
            pltpu.VMEM((CHUNK,), jnp.float32),
            pltpu.VMEM((CHUNK,), jnp.float32),
            pltpu.VMEM((WTAB,), jnp.float32),
            pltpu.VMEM_SHARED((ACC_ROWS, HID), jnp.float32),
            pltpu.VMEM_SHARED((ACC_ROWS,), jnp.float32),
            pltpu.VMEM_SHARED((DW_LEN,), jnp.float32),
            pltpu.SemaphoreType.DMA,
            pltpu.SemaphoreType.DMA,
        ],
    )(_sc_a_body)


def _zero_acc_slice(z128, zbuf, acc_f, base):
    pltpu.sync_copy(z128, zbuf)
    nfull = ROWS_PER_TILE // CHUNK
    rem = ROWS_PER_TILE - nfull * CHUNK
    for k in range(nfull):
        pltpu.sync_copy(zbuf, acc_f.at[pl.ds(base + k * CHUNK, CHUNK)])
    if rem:
        pltpu.sync_copy(zbuf.at[pl.ds(0, rem)],
                        acc_f.at[pl.ds(base + nfull * CHUNK, rem)])


def _sc_a_body(table, w_tab, gidx, sidx, onesc, z128, zsc, zdw,
               out_f, out_bc, out_dw,
               gidx_v, sidx_v, bufs, vones, vw, w_v,
               acc_f, acc_bc, acc_dw, sem0, sem1):
    c = lax.axis_index("c")
    s = lax.axis_index("s")
    base = s * ROWS_PER_TILE
    sems = (sem0, sem1)
    pltpu.sync_copy(onesc, vones)

    @pl.when(s == 0)
    def _():
        pltpu.sync_copy(zdw, acc_dw)

    for v in range(3):
        pltpu.sync_copy(w_tab.at[pl.ds(v * WTAB, WTAB)], w_v)
        _zero_acc_slice(z128, bufs.at[0], acc_f, base)

        @pl.when(s == 0)
        def _():
            pltpu.sync_copy(zsc, acc_bc)

        plsc.subcore_barrier()

        def body(t, carry):
            pltpu.sync_copy(gidx.at[v, c, s, pl.ds(t * SLAB, SLAB)],
                            gidx_v)
            pltpu.sync_copy(sidx.at[v, c, s, pl.ds(t * SLAB, SLAB)],
                            sidx_v)
            cp = pltpu.async_copy(table.at[gidx_v.at[0]], bufs.at[0], sem0)
            for j in range(SLAB):
                p = j % 2
                if j + 1 < SLAB:
                    cp_next = pltpu.async_copy(table.at[gidx_v.at[j + 1]],
                                               bufs.at[1 - p], sems[1 - p])
                cp.wait()
                pltpu.sync_copy(bufs.at[p], acc_f.at[sidx_v.at[j]], add=True)
                for k in range(CHUNK // L):
                    gi = sidx_v[j, pl.ds(k * L, L)]
                    vw[pl.ds(k * L, L)] = plsc.load_gather(w_v, [gi])
                pltpu.sync_copy(vones, acc_bc.at[sidx_v.at[j]], add=True)
                pltpu.sync_copy(vw, acc_dw.at[gidx_v.at[j]], add=True)
                if j + 1 < SLAB:
                    cp = cp_next
            return carry

        nslab = jnp.where(c == 0, T0 // SLAB, T1 // SLAB)
        lax.fori_loop(0, nslab, body, 0)
        plsc.subcore_barrier()
        pltpu.sync_copy(acc_f.at[pl.ds(base, ROWS_PER_TILE)],
                        out_f.at[v, c, pl.ds(base, ROWS_PER_TILE)])

        @pl.when(s == 0)
        def _():
            pltpu.sync_copy(acc_bc, out_bc.at[v, c])

    @pl.when(s == 0)
    def _():
        pltpu.sync_copy(acc_dw, out_dw.at[c])


@functools.cache
def _make_sc_pass_b():
    return functools.partial(
        pl.kernel,
        mesh=plsc.VectorSubcoreMesh(core_axis_name="c", subcore_axis_name="s"),
        compiler_params=pltpu.CompilerParams(needs_layout_passes=False),
        out_type=[
            jax.ShapeDtypeStruct((3, NC, ACC_ROWS, HID), jnp.float32),
        ],
        scratch_types=[
            pltpu.VMEM((SLAB, CHUNK), jnp.int32),
            pltpu.VMEM((SLAB, CHUNK), jnp.int32),
            pltpu.VMEM((2, CHUNK, HID), jnp.float32),
            pltpu.VMEM_SHARED((ACC_ROWS, HID), jnp.float32),
            pltpu.SemaphoreType.DMA,
            pltpu.SemaphoreType.DMA,
        ],
    )(_sc_b_body)


def _sc_b_body(table, gidx, sidx, z128, out_f,
               gidx_v, sidx_v, bufs, acc_f, sem0, sem1):
    c = lax.axis_index("c")
    s = lax.axis_index("s")
    base = s * ROWS_PER_TILE
    sems = (sem0, sem1)
    for v in range(3):
        _zero_acc_slice(z128, bufs.at[0], acc_f, base)
        plsc.subcore_barrier()

        def body(t, carry):
            pltpu.sync_copy(gidx.at[v, c, s, pl.ds(t * SLAB, SLAB)],
                            gidx_v)
            pltpu.sync_copy(sidx.at[v, c, s, pl.ds(t * SLAB, SLAB)],
                            sidx_v)
            cp = pltpu.async_copy(table.at[gidx_v.at[0]], bufs.at[0], sem0)
            for j in range(SLAB):
                p = j % 2
                if j + 1 < SLAB:
                    cp_next = pltpu.async_copy(table.at[gidx_v.at[j + 1]],
                                               bufs.at[1 - p], sems[1 - p])
                cp.wait()
                pltpu.sync_copy(bufs.at[p], acc_f.at[sidx_v.at[j]], add=True)
                if j + 1 < SLAB:
                    cp = cp_next
            return carry

        nslab = jnp.where(c == 0, T0 // SLAB, T1 // SLAB)
        lax.fori_loop(0, nslab, body, 0)
        plsc.subcore_barrier()
        pltpu.sync_copy(acc_f.at[pl.ds(base, ROWS_PER_TILE)],
                        out_f.at[v, c, pl.ds(base, ROWS_PER_TILE)])


def _pre_body(x_ref, w_ref, out_ref):
    out_ref[...] = jnp.dot(x_ref[...], w_ref[0],
                           preferred_element_type=jnp.float32)


def _tc_pre(x, w_stacked):
    return pl.pallas_call(
        _pre_body,
        grid=(3, NB),
        in_specs=[
            pl.BlockSpec((RB, HID), lambda v, i: (i, 0)),
            pl.BlockSpec((1, HID, HID), lambda v, i: (v, 0, 0)),
        ],
        out_specs=pl.BlockSpec((RB, HID), lambda v, i: (v * NB + i, 0)),
        out_shape=jax.ShapeDtypeStruct((3 * N, HID), jnp.float32),
    )(x, w_stacked)


def _mid_body(accf_ref, accs_ref, w_ref, out_ref):
    a = accf_ref[0, 0] + accf_ref[0, 1]
    bc = accs_ref[0, 0] + accs_ref[0, 1]
    binv = jnp.where(bc > 0, 1.0 / bc, 0.0)
    out_ref[...] = (binv * w_ref[...]) * a


def _tc_mid(acc_f, acc_s, w_col):
    return pl.pallas_call(
        _mid_body,
        grid=(3, NB),
        in_specs=[
            pl.BlockSpec((1, NC, RB, HID), lambda v, i: (v, 0, i, 0)),
            pl.BlockSpec((1, NC, RB, 1), lambda v, i: (v, 0, i, 0)),
            pl.BlockSpec((RB, 1), lambda v, i: (v * NB + i, 0)),
        ],
        out_specs=pl.BlockSpec((RB, HID), lambda v, i: (v * NB + i, 0)),
        out_shape=jax.ShapeDtypeStruct((3 * N, HID), jnp.float32),
    )(acc_f, acc_s, w_col)


def _leaky(h):
    return jnp.where(h >= 0, h, 0.2 * h)


def _ln(h, g, b):
    mu = jnp.mean(h, axis=-1, keepdims=True)
    var = jnp.mean((h - mu) ** 2, axis=-1, keepdims=True)
    return (h - mu) * lax.rsqrt(var + 1e-5) * g + b


def _final_body(accf_ref, accs_ref, x_ref, bst_ref,
                wa1_ref, ba1_ref, wa2_ref, ba2_ref,
                wb_ref, bb_ref, gb_ref, beb_ref,
                wg1_ref, bg1_ref, wg2_ref, bg2_ref,
                wc1_ref, bc1_ref, gc_ref, bec_ref, wc2_ref, bc2_ref,
                out_ref):
    x = x_ref[...]
    f32 = jnp.float32
    zs = []
    for v in range(3):
        av = accf_ref[v, 0] + accf_ref[v, 1]
        dw = accs_ref[v, 0] + accs_ref[v, 1]
        dinv = jnp.where(dw > 0, 1.0 / dw, 0.0)
        zs.append(_leaky(dinv * av + bst_ref[v:v + 1]))
    scores = []
    for v in range(3):
        t = jnp.tanh(jnp.dot(zs[v], wa1_ref[...], preferred_element_type=f32)
                     + ba1_ref[...])
        scores.append(jnp.dot(t, wa2_ref[...], preferred_element_type=f32)
                      + ba2_ref[...])
    mx = jnp.maximum(scores[0], jnp.maximum(scores[1], scores[2]))
    es = [jnp.exp(sv - mx) for sv in scores]
    den = es[0] + es[1] + es[2]
    z_sp = (es[0] * zs[0] + es[1] * zs[1] + es[2] * zs[2]) / den
    zb = _leaky(_ln(jnp.dot(x, wb_ref[...], preferred_element_type=f32)
                    + bb_ref[...], gb_ref[...], beb_ref[...]))
    se = jnp.dot(z_sp, wg1_ref[0], preferred_element_type=f32) \
        + jnp.dot(zb, wg1_ref[1], preferred_element_type=f32) + bg1_ref[...]
    se = jnp.maximum(se, 0.0)
    gl = jnp.dot(se, wg2_ref[...], preferred_element_type=f32) + bg2_ref[...]
    g = 1.0 / (1.0 + jnp.exp(-gl))
    zf = g * z_sp + (1.0 - g) * zb
    h = jnp.dot(zf, wc1_ref[0], preferred_element_type=f32) \
        + jnp.dot(x, wc1_ref[1], preferred_element_type=f32) + bc1_ref[...]
    h = jnp.maximum(_ln(h, gc_ref[...], bec_ref[...]), 0.0)
    out_ref[...] = jnp.dot(h, wc2_ref[...], preferred_element_type=f32) \
        + bc2_ref[...]


def _tc_final(acc_f, acc_s, x, bst, wa1, ba1, wa2, ba2, wb, bb, gb, beb,
              wg1, bg1, wg2, bg2, wc1, bc1, gc, bec, wc2, bc2):
    def full(shape):
        return pl.BlockSpec(shape, lambda i: tuple(0 for _ in shape))
    return pl.pallas_call(
        _final_body,
        grid=(NB,),
        in_specs=[
            pl.BlockSpec((3, NC, RB, HID), lambda i: (0, 0, i, 0)),
            pl.BlockSpec((3, NC, RB, 1), lambda i: (0, 0, i, 0)),
            pl.BlockSpec((RB, HID), lambda i: (i, 0)),
            full(bst.shape), full(wa1.shape), full(ba1.shape),
            full(wa2.shape), full(ba2.shape),
            full(wb.shape), full(bb.shape), full(gb.shape), full(beb.shape),
            full(wg1.shape), full(bg1.shape), full(wg2.shape),
            full(bg2.shape),
            full(wc1.shape), full(bc1.shape), full(gc.shape), full(bec.shape),
            full(wc2.shape), full(bc2.shape),
        ],
        out_specs=pl.BlockSpec((RB, 1), lambda i: (i, 0)),
        out_shape=jax.ShapeDtypeStruct((N, 1), jnp.float32),
    )(acc_f, acc_s, x, bst, wa1, ba1, wa2, ba2, wb, bb, gb, beb,
      wg1, bg1, wg2, bg2, wc1, bc1, gc, bec, wc2, bc2)


def _pack_idx(rows, pad_value):
    r = rows.astype(jnp.int32)
    s0 = NS * T0 * CHUNK

    def part(seg, t):
        cap = NS * t * CHUNK
        pad = jnp.full((3, cap - seg.shape[1]), pad_value, jnp.int32)
        a = jnp.concatenate([seg, pad], axis=1).reshape(3, NS, t, CHUNK)
        return jnp.pad(a, ((0, 0), (0, 0), (0, NCHUNK - t), (0, 0)),
                       constant_values=pad_value)

    return jnp.stack([part(r[:, :s0], T0), part(r[:, s0:], T1)], axis=1)


def kernel(x, e_auc, e_ip, e_dev, w_auc, w_ip, w_dev, W_auc, b_auc, W_ip,
           b_ip, W_dev, b_dev, Wa1, ba1, Wa2, ba2, Wb, bb, gb, beb, Wg1, bg1,
           Wg2, bg2, Wc1, bc1, gc, bec, Wc2, bc2):
    f32 = jnp.float32
    w_stacked = jnp.stack([W_auc, W_ip, W_dev])
    bst = jnp.stack([b_auc, b_ip, b_dev])
    ws = jnp.stack([w_auc, w_ip, w_dev])
    w_flat = ws.reshape(3 * N)
    w_tab = jnp.pad(ws, ((0, 0), (0, WTAB - N))).reshape(3 * WTAB)
    onesc = jnp.ones((CHUNK,), f32)
    z128 = jnp.zeros((CHUNK, HID), f32)
    zsc = jnp.zeros((ACC_ROWS,), f32)
    zdw = jnp.zeros((DW_LEN,), f32)

    ni = jnp.stack([e_auc[0], e_ip[0], e_dev[0]])
    ei = jnp.stack([e_auc[1], e_ip[1], e_dev[1]])
    voff = (jnp.arange(3, dtype=jnp.int32) * N)[:, None]
    g_a = _pack_idx(ni + voff, 0)
    s_a = _pack_idx(ei, TRASH)
    g_b = _pack_idx(ei + voff, 0)
    s_b = _pack_idx(ni, TRASH)

    xt = _tc_pre(x, w_stacked)
    accf_a, acc_bc, acc_dw = _make_sc_pass_a()(
        xt, w_tab, g_a, s_a, onesc, z128, zsc, zdw)
    m_t = _tc_mid(accf_a, acc_bc.reshape(3, NC, ACC_ROWS, 1),
                  w_flat.reshape(3 * N, 1))
    accf_b, = _make_sc_pass_b()(m_t, g_b, s_b, z128)
    dw3 = acc_dw.reshape(NC, 3, N).transpose(1, 0, 2)[..., None]
    return _tc_final(
        accf_b, dw3, x, bst,
        Wa1, ba1.reshape(1, -1), Wa2, ba2.reshape(1, 1),
        Wb, bb.reshape(1, -1), gb.reshape(1, -1), beb.reshape(1, -1),
        Wg1.reshape(2, HID, HID // 2), bg1.reshape(1, -1), Wg2,
        bg2.reshape(1, -1),
        Wc1.reshape(2, HID, HID), bc1.reshape(1, -1), gc.reshape(1, -1),
        bec.reshape(1, -1), Wc2, bc2.reshape(1, 1))


# speedup vs baseline: 1.4901x; 1.0237x over previous
import functools

import jax
import jax.numpy as jnp
from jax import lax
from jax.experimental import pallas as pl
from jax.experimental.pallas import tpu as pltpu
from jax.experimental.pallas import tpu_sc as plsc

N = 10000
E = 320000
HID = 128
NC = 2
NS = 16
L = 16
CHUNK = 128
T0 = 152
T1 = 8
NCHUNK = max(T0, T1)
SLAB = 8
ROWS_PER_TILE = 632
ACC_ROWS = NS * ROWS_PER_TILE
TRASH = N
WTAB = N + 128
DW_LEN = 3 * N
RB = 400
NB = N // RB


@functools.cache
def _make_sc_pass_a():
    return functools.partial(
        pl.kernel,
        mesh=plsc.VectorSubcoreMesh(core_axis_name="c", subcore_axis_name="s"),
        compiler_params=pltpu.CompilerParams(needs_layout_passes=False),
        out_type=[
            jax.ShapeDtypeStruct((3, NC, ACC_ROWS, HID), jnp.float32),
            jax.ShapeDtypeStruct((3, NC, ACC_ROWS), jnp.float32),
            jax.ShapeDtypeStruct((NC, DW_LEN), jnp.float32),
        ],
        scratch_types=[
            pltpu.VMEM((SLAB, CHUNK), jnp.int32),
            pltpu.VMEM((SLAB, CHUNK), jnp.int32),
            pltpu.VMEM((2, CHUNK, HID), jnp.float32),
            pltpu.VMEM((CHUNK,), jnp.float32),
            pltpu.VMEM((CHUNK,), jnp.float32),
            pltpu.VMEM((WTAB,), jnp.float32),
            pltpu.VMEM_SHARED((ACC_ROWS, HID), jnp.float32),
            pltpu.VMEM_SHARED((ACC_ROWS,), jnp.float32),
            pltpu.VMEM_SHARED((DW_LEN,), jnp.float32),
            pltpu.SemaphoreType.DMA,
            pltpu.SemaphoreType.DMA,
        ],
    )(_sc_a_body)


def _zero_acc_slice(z128, zbuf, acc_f, base):
    pltpu.sync_copy(z128, zbuf)
    nfull = ROWS_PER_TILE // CHUNK
    rem = ROWS_PER_TILE - nfull * CHUNK
    for k in range(nfull):
        pltpu.sync_copy(zbuf, acc_f.at[pl.ds(base + k * CHUNK, CHUNK)])
    if rem:
        pltpu.sync_copy(zbuf.at[pl.ds(0, rem)],
                        acc_f.at[pl.ds(base + nfull * CHUNK, rem)])


def _sc_a_body(table, w_tab, gidx, sidx, onesc, z128, zsc, zdw,
               out_f, out_bc, out_dw,
               gidx_v, sidx_v, bufs, vones, vw, w_v,
               acc_f, acc_bc, acc_dw, sem0, sem1):
    c = lax.axis_index("c")
    s = lax.axis_index("s")
    base = s * ROWS_PER_TILE
    sems = (sem0, sem1)
    pltpu.sync_copy(onesc, vones)

    @pl.when(s == 0)
    def _():
        pltpu.sync_copy(zdw, acc_dw)

    for v in range(3):
        pltpu.sync_copy(w_tab.at[pl.ds(v * WTAB, WTAB)], w_v)
        _zero_acc_slice(z128, bufs.at[0], acc_f, base)

        @pl.when(s == 0)
        def _():
            pltpu.sync_copy(zsc, acc_bc)

        plsc.subcore_barrier()

        def body(t, carry):
            pltpu.sync_copy(gidx.at[v, c, s, pl.ds(t * SLAB, SLAB)],
                            gidx_v)
            pltpu.sync_copy(sidx.at[v, c, s, pl.ds(t * SLAB, SLAB)],
                            sidx_v)
            cp = pltpu.async_copy(table.at[gidx_v.at[0]], bufs.at[0], sem0)
            for j in range(SLAB):
                p = j % 2
                if j + 1 < SLAB:
                    cp_next = pltpu.async_copy(table.at[gidx_v.at[j + 1]],
                                               bufs.at[1 - p], sems[1 - p])
                cp.wait()
                pltpu.sync_copy(bufs.at[p], acc_f.at[sidx_v.at[j]], add=True)
                for k in range(CHUNK // L):
                    gi = sidx_v[j, pl.ds(k * L, L)]
                    vw[pl.ds(k * L, L)] = plsc.load_gather(w_v, [gi])
                pltpu.sync_copy(vones, acc_bc.at[sidx_v.at[j]], add=True)
                pltpu.sync_copy(vw, acc_dw.at[gidx_v.at[j]], add=True)
                if j + 1 < SLAB:
                    cp = cp_next
            return carry

        nslab = jnp.where(c == 0, T0 // SLAB, T1 // SLAB)
        lax.fori_loop(0, nslab, body, 0)
        plsc.subcore_barrier()
        pltpu.sync_copy(acc_f.at[pl.ds(base, ROWS_PER_TILE)],
                        out_f.at[v, c, pl.ds(base, ROWS_PER_TILE)])

        @pl.when(s == 0)
        def _():
            pltpu.sync_copy(acc_bc, out_bc.at[v, c])

    @pl.when(s == 0)
    def _():
        pltpu.sync_copy(acc_dw, out_dw.at[c])


@functools.cache
def _make_sc_pass_b():
    return functools.partial(
        pl.kernel,
        mesh=plsc.VectorSubcoreMesh(core_axis_name="c", subcore_axis_name="s"),
        compiler_params=pltpu.CompilerParams(needs_layout_passes=False),
        out_type=[
            jax.ShapeDtypeStruct((3, NC, ACC_ROWS, HID), jnp.float32),
        ],
        scratch_types=[
            pltpu.VMEM((SLAB, CHUNK), jnp.int32),
            pltpu.VMEM((SLAB, CHUNK), jnp.int32),
            pltpu.VMEM((2, CHUNK, HID), jnp.float32),
            pltpu.VMEM_SHARED((ACC_ROWS, HID), jnp.float32),
            pltpu.SemaphoreType.DMA,
            pltpu.SemaphoreType.DMA,
        ],
    )(_sc_b_body)


def _sc_b_body(table, gidx, sidx, z128, out_f,
               gidx_v, sidx_v, bufs, acc_f, sem0, sem1):
    c = lax.axis_index("c")
    s = lax.axis_index("s")
    base = s * ROWS_PER_TILE
    sems = (sem0, sem1)
    for v in range(3):
        _zero_acc_slice(z128, bufs.at[0], acc_f, base)
        plsc.subcore_barrier()

        def body(t, carry):
            pltpu.sync_copy(gidx.at[v, c, s, pl.ds(t * SLAB, SLAB)],
                            gidx_v)
            pltpu.sync_copy(sidx.at[v, c, s, pl.ds(t * SLAB, SLAB)],
                            sidx_v)
            cp = pltpu.async_copy(table.at[gidx_v.at[0]], bufs.at[0], sem0)
            for j in range(SLAB):
                p = j % 2
                if j + 1 < SLAB:
                    cp_next = pltpu.async_copy(table.at[gidx_v.at[j + 1]],
                                               bufs.at[1 - p], sems[1 - p])
                cp.wait()
                pltpu.sync_copy(bufs.at[p], acc_f.at[sidx_v.at[j]], add=True)
                if j + 1 < SLAB:
                    cp = cp_next
            return carry

        nslab = jnp.where(c == 0, T0 // SLAB, T1 // SLAB)
        lax.fori_loop(0, nslab, body, 0)
        plsc.subcore_barrier()
        pltpu.sync_copy(acc_f.at[pl.ds(base, ROWS_PER_TILE)],
                        out_f.at[v, c, pl.ds(base, ROWS_PER_TILE)])


def _pre_body(x_ref, w_ref, out_ref):
    out_ref[...] = jnp.dot(x_ref[...], w_ref[0],
                           preferred_element_type=jnp.float32)


def _tc_pre(x, w_stacked):
    return pl.pallas_call(
        _pre_body,
        grid=(3, NB),
        in_specs=[
            pl.BlockSpec((RB, HID), lambda v, i: (i, 0)),
            pl.BlockSpec((1, HID, HID), lambda v, i: (v, 0, 0)),
        ],
        out_specs=pl.BlockSpec((RB, HID), lambda v, i: (v * NB + i, 0)),
        out_shape=jax.ShapeDtypeStruct((3 * N, HID), jnp.float32),
    )(x, w_stacked)


def _mid_body(accf_ref, accs_ref, w_ref, out_ref):
    a = accf_ref[0, 0] + accf_ref[0, 1]
    bc = accs_ref[0, 0] + accs_ref[0, 1]
    binv = jnp.where(bc > 0, 1.0 / bc, 0.0)
    out_ref[...] = (binv * w_ref[...]) * a


def _tc_mid(acc_f, acc_s, w_col):
    return pl.pallas_call(
        _mid_body,
        grid=(3, NB),
        in_specs=[
            pl.BlockSpec((1, NC, RB, HID), lambda v, i: (v, 0, i, 0)),
            pl.BlockSpec((1, NC, RB, 1), lambda v, i: (v, 0, i, 0)),
            pl.BlockSpec((RB, 1), lambda v, i: (v * NB + i, 0)),
        ],
        out_specs=pl.BlockSpec((RB, HID), lambda v, i: (v * NB + i, 0)),
        out_shape=jax.ShapeDtypeStruct((3 * N, HID), jnp.float32),
    )(acc_f, acc_s, w_col)


def _leaky(h):
    return jnp.where(h >= 0, h, 0.2 * h)


def _ln(h, g, b):
    mu = jnp.mean(h, axis=-1, keepdims=True)
    var = jnp.mean((h - mu) ** 2, axis=-1, keepdims=True)
    return (h - mu) * lax.rsqrt(var + 1e-5) * g + b


def _final_body(accf_ref, accs_ref, x_ref, bst_ref,
                wa1_ref, ba1_ref, wa2_ref, ba2_ref,
                wb_ref, bb_ref, gb_ref, beb_ref,
                wg1_ref, bg1_ref, wg2_ref, bg2_ref,
                wc1_ref, bc1_ref, gc_ref, bec_ref, wc2_ref, bc2_ref,
                out_ref):
    x = x_ref[...]
    f32 = jnp.float32
    zs = []
    for v in range(3):
        av = accf_ref[v, 0] + accf_ref[v, 1]
        dw = accs_ref[v, 0] + accs_ref[v, 1]
        dinv = jnp.where(dw > 0, 1.0 / dw, 0.0)
        zs.append(_leaky(dinv * av + bst_ref[v:v + 1]))
    scores = []
    for v in range(3):
        t = jnp.tanh(jnp.dot(zs[v], wa1_ref[...], preferred_element_type=f32)
                     + ba1_ref[...])
        scores.append(jnp.dot(t, wa2_ref[...], preferred_element_type=f32)
                      + ba2_ref[...])
    mx = jnp.maximum(scores[0], jnp.maximum(scores[1], scores[2]))
    es = [jnp.exp(sv - mx) for sv in scores]
    den = es[0] + es[1] + es[2]
    z_sp = (es[0] * zs[0] + es[1] * zs[1] + es[2] * zs[2]) / den
    zb = _leaky(_ln(jnp.dot(x, wb_ref[...], preferred_element_type=f32)
                    + bb_ref[...], gb_ref[...], beb_ref[...]))
    se = jnp.dot(z_sp, wg1_ref[0], preferred_element_type=f32) \
        + jnp.dot(zb, wg1_ref[1], preferred_element_type=f32) + bg1_ref[...]
    se = jnp.maximum(se, 0.0)
    gl = jnp.dot(se, wg2_ref[...], preferred_element_type=f32) + bg2_ref[...]
    g = 1.0 / (1.0 + jnp.exp(-gl))
    zf = g * z_sp + (1.0 - g) * zb
    h = jnp.dot(zf, wc1_ref[0], preferred_element_type=f32) \
        + jnp.dot(x, wc1_ref[1], preferred_element_type=f32) + bc1_ref[...]
    h = jnp.maximum(_ln(h, gc_ref[...], bec_ref[...]), 0.0)
    out_ref[...] = jnp.dot(h, wc2_ref[...], preferred_element_type=f32) \
        + bc2_ref[...]


def _tc_final(acc_f, acc_s, x, bst, wa1, ba1, wa2, ba2, wb, bb, gb, beb,
              wg1, bg1, wg2, bg2, wc1, bc1, gc, bec, wc2, bc2):
    def full(shape):
        return pl.BlockSpec(shape, lambda i: tuple(0 for _ in shape))
    return pl.pallas_call(
        _final_body,
        grid=(NB,),
        in_specs=[
            pl.BlockSpec((3, NC, RB, HID), lambda i: (0, 0, i, 0)),
            pl.BlockSpec((3, NC, RB, 1), lambda i: (0, 0, i, 0)),
            pl.BlockSpec((RB, HID), lambda i: (i, 0)),
            full(bst.shape), full(wa1.shape), full(ba1.shape),
            full(wa2.shape), full(ba2.shape),
            full(wb.shape), full(bb.shape), full(gb.shape), full(beb.shape),
            full(wg1.shape), full(bg1.shape), full(wg2.shape),
            full(bg2.shape),
            full(wc1.shape), full(bc1.shape), full(gc.shape), full(bec.shape),
            full(wc2.shape), full(bc2.shape),
        ],
        out_specs=pl.BlockSpec((RB, 1), lambda i: (i, 0)),
        out_shape=jax.ShapeDtypeStruct((N, 1), jnp.float32),
    )(acc_f, acc_s, x, bst, wa1, ba1, wa2, ba2, wb, bb, gb, beb,
      wg1, bg1, wg2, bg2, wc1, bc1, gc, bec, wc2, bc2)


def _pack_idx(rows, pad_value):
    r = rows.astype(jnp.int32)
    s0 = NS * T0 * CHUNK

    def part(seg, t):
        cap = NS * t * CHUNK
        pad = jnp.full((3, cap - seg.shape[1]), pad_value, jnp.int32)
        a = jnp.concatenate([seg, pad], axis=1).reshape(3, NS, t, CHUNK)
        return jnp.pad(a, ((0, 0), (0, 0), (0, NCHUNK - t), (0, 0)),
                       constant_values=pad_value)

    return jnp.stack([part(r[:, :s0], T0), part(r[:, s0:], T1)], axis=1)


def kernel(x, e_auc, e_ip, e_dev, w_auc, w_ip, w_dev, W_auc, b_auc, W_ip,
           b_ip, W_dev, b_dev, Wa1, ba1, Wa2, ba2, Wb, bb, gb, beb, Wg1, bg1,
           Wg2, bg2, Wc1, bc1, gc, bec, Wc2, bc2):
    f32 = jnp.float32
    w_stacked = jnp.stack([W_auc, W_ip, W_dev])
    bst = jnp.stack([b_auc, b_ip, b_dev])
    ws = jnp.stack([w_auc, w_ip, w_dev])
    w_flat = ws.reshape(3 * N)
    w_tab = jnp.pad(ws, ((0, 0), (0, WTAB - N))).reshape(3 * WTAB)
    onesc = jnp.ones((CHUNK,), f32)
    z128 = jnp.zeros((CHUNK, HID), f32)
    zsc = jnp.zeros((ACC_ROWS,), f32)
    zdw = jnp.zeros((DW_LEN,), f32)

    ni = jnp.stack([e_auc[0], e_ip[0], e_dev[0]])
    ei = jnp.stack([e_auc[1], e_ip[1], e_dev[1]])
    voff = (jnp.arange(3, dtype=jnp.int32) * N)[:, None]
    g_a = _pack_idx(ni + voff, 0)
    s_a = _pack_idx(ei, TRASH)
    g_b = _pack_idx(ei + voff, 0)
    s_b = _pack_idx(ni, TRASH)

    xt = _tc_pre(x, w_stacked)
    accf_a, acc_bc, acc_dw = _make_sc_pass_a()(
        xt, w_tab, g_a, s_a, onesc, z128, zsc, zdw)
    m_t = _tc_mid(accf_a, acc_bc.reshape(3, NC, ACC_ROWS, 1),
                  w_flat.reshape(3 * N, 1))
    accf_b, = _make_sc_pass_b()(m_t, g_b, s_b, z128)
    dw3 = acc_dw.reshape(NC, 3, N).transpose(1, 0, 2)[..., None]
    return _tc_final(
        accf_b, dw3, x, bst,
        Wa1, ba1.reshape(1, -1), Wa2, ba2.reshape(1, 1),
        Wb, bb.reshape(1, -1), gb.reshape(1, -1), beb.reshape(1, -1),
        Wg1.reshape(2, HID, HID // 2), bg1.reshape(1, -1), Wg2,
        bg2.reshape(1, -1),
        Wc1.reshape(2, HID, HID), bc1.reshape(1, -1), gc.reshape(1, -1),
        bec.reshape(1, -1), Wc2, bc2.reshape(1, 1))
